# Initial kernel scaffold; baseline (speedup 1.0000x reference)
import jax
import jax.numpy as jnp
from jax.experimental import pallas as pl


def kernel(x, edge_index, enc_w, enc_b, att_w, a_src, a_dst, dec_w, dec_b, ks_raw):
    raise NotImplementedError("write your pallas kernel here")


# speedup vs baseline: 3.6036x; 3.6036x over previous
import math

import jax
import jax.numpy as jnp
from jax import lax
from jax.experimental import pallas as pl
from jax.experimental.pallas import tpu as pltpu
from jax.experimental.pallas import tpu_sc as plsc

N = 10000
E = 320000
F = 128
H = 8
NCLASS = 40
NLAYERS = 4
PI = math.pi

NC, NS, LANES = 2, 16, 16
NW = NC * NS
EPW = E // NW
NP = 10240
RPT = NP // NS
RCH = 128

RB = 1000
GN = N // RB
RB2 = 80
GN2 = N // RB2
NPB = NP // RB2

B1 = 80
NB1 = EPW // B1
B2 = 80
NB2 = EPW // B2
BS = 80
NBS = EPW // BS

f32 = jnp.float32
i32 = jnp.int32

_MESH = plsc.VectorSubcoreMesh(core_axis_name="c", subcore_axis_name="s")


def _fold_body(aw_ref, sel_ref, o_ref):
    o_ref[...] = jnp.dot(aw_ref[...], sel_ref[...], preferred_element_type=f32)


def _fold(att_w, asel):
    return pl.pallas_call(
        _fold_body,
        out_shape=jax.ShapeDtypeStruct((F, 2 * H), f32),
    )(att_w, asel)


def _front_body(x_ref, ew_ref, eb_ref, wsd_ref,
                y_ref, om_ref, vc_ref, vs_ref, as_ref, ad_ref, ml_ref):
    yb = jnp.maximum(
        jnp.dot(x_ref[...], ew_ref[...], preferred_element_type=f32)
        + eb_ref[...], 0.0)
    y_ref[...] = yb
    om_ref[...] = jnp.clip(yb, 0.0, PI)
    vc_ref[...] = jnp.cos(yb)
    vs_ref[...] = jnp.sin(yb)
    asd = jnp.dot(yb, wsd_ref[...], preferred_element_type=f32)
    as_ref[...] = jnp.concatenate([asd[:, :H], asd[:, :H]], axis=1)
    ad_ref[...] = jnp.concatenate([asd[:, H:], asd[:, H:]], axis=1)
    m = jnp.max(asd, axis=0, keepdims=True)

    @pl.when(pl.program_id(0) == 0)
    def _():
        ml_ref[...] = m

    @pl.when(pl.program_id(0) != 0)
    def _():
        mm = jnp.maximum(ml_ref[...], m)
        ml_ref[...] = mm

    @pl.when(pl.program_id(0) == GN - 1)
    def _():
        mm = ml_ref[...]
        s = mm[:, :H] + mm[:, H:]
        s = jnp.where(s > 0, s, 0.2 * s)
        ml_ref[...] = jnp.concatenate([s, s], axis=1)


def _front(x, enc_w, eb2, wsd):
    return pl.pallas_call(
        _front_body,
        grid=(GN,),
        in_specs=[
            pl.BlockSpec((RB, F), lambda i: (i, 0)),
            pl.BlockSpec((F, F), lambda i: (0, 0)),
            pl.BlockSpec((1, F), lambda i: (0, 0)),
            pl.BlockSpec((F, 2 * H), lambda i: (0, 0)),
        ],
        out_specs=[
            pl.BlockSpec((RB, F), lambda i: (i, 0)),
            pl.BlockSpec((RB, F), lambda i: (i, 0)),
            pl.BlockSpec((RB, F), lambda i: (i, 0)),
            pl.BlockSpec((RB, F), lambda i: (i, 0)),
            pl.BlockSpec((RB, 2 * H), lambda i: (i, 0)),
            pl.BlockSpec((RB, 2 * H), lambda i: (i, 0)),
            pl.BlockSpec((1, 2 * H), lambda i: (0, 0)),
        ],
        out_shape=[
            jax.ShapeDtypeStruct((N, F), f32),
            jax.ShapeDtypeStruct((N, F), f32),
            jax.ShapeDtypeStruct((N, F), f32),
            jax.ShapeDtypeStruct((N, F), f32),
            jax.ShapeDtypeStruct((N, 2 * H), f32),
            jax.ShapeDtypeStruct((N, 2 * H), f32),
            jax.ShapeDtypeStruct((1, 2 * H), f32),
        ],
    )(x, enc_w, eb2, wsd)


def _dinv_body(d0_ref, d1_ref, o_ref):
    inv = 1.0 / (d0_ref[:, :H] + d1_ref[:, :H] + 1e-16)
    o_ref[...] = jnp.concatenate([inv, inv], axis=1)


def _dinv(dparts):
    return pl.pallas_call(
        _dinv_body,
        grid=(GN2,),
        in_specs=[
            pl.BlockSpec((RB2, 2 * H), lambda i: (i, 0)),
            pl.BlockSpec((RB2, 2 * H), lambda i: (i + NPB, 0)),
        ],
        out_specs=pl.BlockSpec((RB2, 2 * H), lambda i: (i, 0)),
        out_shape=jax.ShapeDtypeStruct((N, 2 * H), f32),
    )(dparts, dparts)


def _update_body(ks_ref, y_ref, om_ref, p0, p1, c0, c1, s0, s1,
                 yn_ref, vc_ref, vs_ref):
    ks = ks_ref[0, 0]
    Ks = jnp.log1p(jnp.exp(ks))
    phi = p0[...] + p1[...]
    cr = c0[...] + c1[...]
    sr = s0[...] + s1[...]
    Rm = jnp.sqrt(cr * cr + sr * sr)
    y = y_ref[...]
    yn = y + om_ref[...] + Ks * Rm * jnp.sin(phi - y)
    yn_ref[...] = yn
    vc_ref[...] = jnp.cos(yn)
    vs_ref[...] = jnp.sin(yn)


def _update(ks2, y, om, pp, cp, sp):
    part0 = pl.BlockSpec((RB2, F), lambda i: (i, 0))
    part1 = pl.BlockSpec((RB2, F), lambda i: (i + NPB, 0))
    return pl.pallas_call(
        _update_body,
        grid=(GN2,),
        in_specs=[
            pl.BlockSpec((1, 1), lambda i: (0, 0)),
            pl.BlockSpec((RB2, F), lambda i: (i, 0)),
            pl.BlockSpec((RB2, F), lambda i: (i, 0)),
            part0, part1, part0, part1, part0, part1,
        ],
        out_specs=[
            pl.BlockSpec((RB2, F), lambda i: (i, 0)),
            pl.BlockSpec((RB2, F), lambda i: (i, 0)),
            pl.BlockSpec((RB2, F), lambda i: (i, 0)),
        ],
        out_shape=[
            jax.ShapeDtypeStruct((N, F), f32),
            jax.ShapeDtypeStruct((N, F), f32),
            jax.ShapeDtypeStruct((N, F), f32),
        ],
    )(ks2, y, om, pp, pp, cp, cp, sp, sp)


def _dec_body(y_ref, w_ref, b_ref, o_ref):
    o_ref[...] = (jnp.dot(y_ref[...], w_ref[...], preferred_element_type=f32)
                  + b_ref[...])


def _dec(y, dwp, dbp):
    return pl.pallas_call(
        _dec_body,
        grid=(GN,),
        in_specs=[
            pl.BlockSpec((RB, F), lambda i: (i, 0)),
            pl.BlockSpec((F, F), lambda i: (0, 0)),
            pl.BlockSpec((1, F), lambda i: (0, 0)),
        ],
        out_specs=pl.BlockSpec((RB, F), lambda i: (i, 0)),
        out_shape=jax.ShapeDtypeStruct((N, F), f32),
    )(y, dwp, dbp)


def _pass1_body(asrc_hbm, adst_hbm, src_hbm, dst_hbm, ml_hbm, z8_hbm,
                p_hbm, dp_hbm,
                acc_sh, is_v, id_v, rs_v, rd_v, pb_v, m_v, cp_v, sem):
    cid = lax.axis_index("c")
    sid = lax.axis_index("s")
    wid = sid * NC + cid
    ebase = wid * EPW
    rbase = sid * RPT

    pltpu.sync_copy(z8_hbm, cp_v)
    for i in range(RPT // RCH):
        pltpu.sync_copy(cp_v, acc_sh.at[pl.ds(rbase + i * RCH, RCH)])
    pltpu.sync_copy(ml_hbm, m_v)
    plsc.subcore_barrier()

    ml = m_v[...]

    def blk(j, carry):
        base = ebase + j * B1
        pltpu.sync_copy(src_hbm.at[pl.ds(base, B1)], is_v)
        pltpu.sync_copy(dst_hbm.at[pl.ds(base, B1)], id_v)
        g1 = pltpu.async_copy(asrc_hbm.at[is_v], rs_v, sem)
        g2 = pltpu.async_copy(adst_hbm.at[id_v], rd_v, sem)
        g1.wait()
        g2.wait()
        for e in range(B1):
            z = rs_v[e, :] + rd_v[e, :]
            zl = jnp.where(z > 0, z, 0.2 * z)
            pb_v[e, :] = jnp.exp(zl - ml)
        pltpu.sync_copy(pb_v, p_hbm.at[pl.ds(base, B1)])
        pltpu.sync_copy(pb_v, acc_sh.at[id_v], add=True)
        return carry

    lax.fori_loop(0, NB1, blk, 0)
    plsc.subcore_barrier()
    for i in range(RPT // RCH):
        pltpu.sync_copy(acc_sh.at[pl.ds(rbase + i * RCH, RCH)], cp_v)
        pltpu.sync_copy(cp_v, dp_hbm.at[pl.ds(cid * NP + rbase + i * RCH, RCH)])


def _pass1(asrc, adst, src, dst, ml, z8):
    return pl.kernel(
        _pass1_body,
        out_type=[
            jax.ShapeDtypeStruct((E, 2 * H), f32),
            jax.ShapeDtypeStruct((2 * NP, 2 * H), f32),
        ],
        mesh=_MESH,
        scratch_types=[
            pltpu.VMEM_SHARED((NP, 2 * H), f32),
            pltpu.VMEM((B1,), i32),
            pltpu.VMEM((B1,), i32),
            pltpu.VMEM((B1, 2 * H), f32),
            pltpu.VMEM((B1, 2 * H), f32),
            pltpu.VMEM((B1, 2 * H), f32),
            pltpu.VMEM((LANES,), f32),
            pltpu.VMEM((RCH, 2 * H), f32),
            pltpu.SemaphoreType.DMA,
        ],
        compiler_params=pltpu.CompilerParams(use_tc_tiling_on_sc=False),
    )(asrc, adst, src, dst, ml, z8)


def _qmul_body(p_hbm, dinv_hbm, dst_hbm, q_hbm,
               id_v, dg_v, pb_v, qb_v, sem):
    cid = lax.axis_index("c")
    sid = lax.axis_index("s")
    wid = sid * NC + cid
    ebase = wid * EPW

    def blk(j, carry):
        base = ebase + j * B2
        pltpu.sync_copy(dst_hbm.at[pl.ds(base, B2)], id_v)
        g = pltpu.async_copy(dinv_hbm.at[id_v], dg_v, sem)
        pltpu.sync_copy(p_hbm.at[pl.ds(base, B2)], pb_v)
        g.wait()
        for e in range(B2):
            qb_v[e, :] = pb_v[e, :] * dg_v[e, :]
        pltpu.sync_copy(qb_v, q_hbm.at[pl.ds(base, B2)])
        return carry

    lax.fori_loop(0, NB2, blk, 0)


def _qmul(p, dinv, dst):
    return pl.kernel(
        _qmul_body,
        out_type=jax.ShapeDtypeStruct((E, 2 * H), f32),
        mesh=_MESH,
        scratch_types=[
            pltpu.VMEM((B2,), i32),
            pltpu.VMEM((B2, 2 * H), f32),
            pltpu.VMEM((B2, 2 * H), f32),
            pltpu.VMEM((B2, 2 * H), f32),
            pltpu.SemaphoreType.DMA,
        ],
        compiler_params=pltpu.CompilerParams(use_tc_tiling_on_sc=False),
    )(p, dinv, dst)


EB = 4000
GE = E // EB


def _attnred_body(q_ref, o_ref):
    o_ref[...] = jnp.sum(q_ref[...], axis=1, keepdims=True) * (1.0 / (2 * H))


def _attnred(q):
    return pl.pallas_call(
        _attnred_body,
        grid=(GE,),
        in_specs=[pl.BlockSpec((EB, 2 * H), lambda i: (i, 0))],
        out_specs=pl.BlockSpec((EB, 1), lambda i: (i, 0)),
        out_shape=jax.ShapeDtypeStruct((E, 1), f32),
    )(q)


def _spmm_body(vy, vc, vs, src_hbm, dst_hbm, attn_hbm, zf_hbm,
               op_y, op_c, op_s,
               acc_sh, is_v, id_v, at_v, g_v, zb_v, dp_v, sem):
    cid = lax.axis_index("c")
    sid = lax.axis_index("s")
    wid = sid * NC + cid
    ebase = wid * EPW
    rbase = sid * RPT
    pltpu.sync_copy(zf_hbm, zb_v)

    for (vin, oout) in ((vy, op_y), (vc, op_c), (vs, op_s)):
        for i in range(RPT // RCH):
            pltpu.sync_copy(zb_v, acc_sh.at[pl.ds(rbase + i * RCH, RCH)])
        plsc.subcore_barrier()

        def blk(j, carry, vin=vin):
            base = ebase + j * BS
            pltpu.sync_copy(src_hbm.at[pl.ds(base, BS)], is_v)
            pltpu.sync_copy(dst_hbm.at[pl.ds(base, BS)], id_v)
            pltpu.sync_copy(attn_hbm.at[pl.ds(base, BS)], at_v)
            pltpu.async_copy(vin.at[id_v], g_v, sem).wait()
            for t2 in range(BS // LANES):
                atv = at_v[pl.ds(t2 * LANES, LANES)]
                for e16 in range(LANES):
                    e = t2 * LANES + e16
                    av = jnp.broadcast_to(atv[e16], (LANES,))
                    for t in range(F // LANES):
                        sl = pl.ds(t * LANES, LANES)
                        g_v[e, sl] = g_v[e, sl] * av
            pltpu.sync_copy(g_v, acc_sh.at[is_v], add=True)
            return carry

        lax.fori_loop(0, NBS, blk, 0)
        plsc.subcore_barrier()
        for i in range(RPT // RCH):
            pltpu.sync_copy(acc_sh.at[pl.ds(rbase + i * RCH, RCH)], dp_v)
            pltpu.sync_copy(
                dp_v, oout.at[pl.ds(cid * NP + rbase + i * RCH, RCH)])


def _spmm(y, vc, vs, src, dst, attn, zf):
    return pl.kernel(
        _spmm_body,
        out_type=[
            jax.ShapeDtypeStruct((2 * NP, F), f32),
            jax.ShapeDtypeStruct((2 * NP, F), f32),
            jax.ShapeDtypeStruct((2 * NP, F), f32),
        ],
        mesh=_MESH,
        scratch_types=[
            pltpu.VMEM_SHARED((NP, F), f32),
            pltpu.VMEM((BS,), i32),
            pltpu.VMEM((BS,), i32),
            pltpu.VMEM((BS,), f32),
            pltpu.VMEM((BS, F), f32),
            pltpu.VMEM((RCH, F), f32),
            pltpu.VMEM((RCH, F), f32),
            pltpu.SemaphoreType.DMA,
        ],
    )(y, vc, vs, src, dst, attn, zf)


def kernel(x, edge_index, enc_w, enc_b, att_w, a_src, a_dst,
           dec_w, dec_b, ks_raw):
    src = edge_index[0]
    dst = edge_index[1]
    mask = jnp.repeat(jnp.eye(H, dtype=f32), F, axis=0)
    asel = jnp.concatenate([mask * a_src.reshape(-1)[:, None],
                            mask * a_dst.reshape(-1)[:, None]], axis=1)
    eb2 = enc_b.reshape(1, F)
    ks2 = ks_raw.reshape(1, 1)
    dwp = jnp.zeros((F, F), f32).at[:, :NCLASS].set(dec_w)
    dbp = jnp.zeros((1, F), f32).at[0, :NCLASS].set(dec_b)
    z8 = jnp.zeros((RCH, 2 * H), f32)
    zf = jnp.zeros((RCH, F), f32)

    wsd = _fold(att_w, asel)
    y, om, vc, vs, asr, adr, ml = _front(x, enc_w, eb2, wsd)
    p, dparts = _pass1(asr, adr, src, dst, ml.reshape(2 * H), z8)
    dinv = _dinv(dparts)
    q = _qmul(p, dinv, dst)
    attn = _attnred(q).reshape(E)
    for _ in range(NLAYERS):
        pp, cp, sp = _spmm(y, vc, vs, src, dst, attn, zf)
        y, vc, vs = _update(ks2, y, om, pp, cp, sp)
    out = _dec(y, dwp, dbp)
    return out[:, :NCLASS]
